# baseline (speedup 1.0000x reference)
import functools

import jax
import jax.numpy as jnp
from jax import lax
from jax.experimental import pallas as pl
from jax.experimental.pallas import tpu as pltpu
from jax.experimental.pallas import tpu_sc as plsc

NC = 2
NS = 16
NW = NC * NS

LANES = 16
NBUF = 8
SPG = 4
FLUSH = 32
STG = 128


def _make_sc_pool(batch, seq, d):
    spw = batch // NW
    gr = SPG * seq
    n_g = spw // SPG
    g_per_flush = FLUSH // SPG
    n_col = d // LANES
    assert n_g % (2 * g_per_flush) == 0
    q_g = 2 * g_per_flush
    n_q = n_g // q_g
    mesh = plsc.VectorSubcoreMesh(core_axis_name="c", subcore_axis_name="s")

    @functools.partial(
        pl.kernel,
        mesh=mesh,
        out_type=jax.ShapeDtypeStruct((batch, d), jnp.float32),
        scratch_types=[
            pltpu.VMEM((spw * seq,), jnp.int32),
            pltpu.VMEM((STG, seq), jnp.int32),
            [pltpu.VMEM((gr, d), jnp.float32) for _ in range(NBUF)],
            [pltpu.VMEM((FLUSH, d), jnp.float32) for _ in range(2)],
            [pltpu.SemaphoreType.DMA for _ in range(NBUF)],
            [pltpu.SemaphoreType.DMA for _ in range(2)],
        ],
    )
    def pool(tbl_hbm, x_hbm, out_hbm, idx_v, stg_v, ring, accs, ring_sems,
             out_sems):
        wid = lax.axis_index("s") * NC + lax.axis_index("c")
        out_base = wid * spw

        def stage_body(part, _):
            pltpu.sync_copy(
                x_hbm.at[pl.ds(out_base + part * STG, STG), :], stg_v)

            def repack_body(r, _):
                p = (part * STG + r) * seq
                v0 = stg_v[r, pl.ds(0, LANES)]
                v1 = stg_v[r, pl.ds(seq - LANES, LANES)]
                idx_v[pl.ds(p, LANES)] = v0
                idx_v[pl.ds(p + seq - LANES, LANES)] = v1
                return 0

            lax.fori_loop(0, STG, repack_body, 0)
            return 0

        lax.fori_loop(0, spw // STG, stage_body, 0)

        def fire(g, slot):
            pltpu.async_copy(
                tbl_hbm.at[idx_v.at[pl.ds(g * gr, gr)]],
                ring[slot], ring_sems[slot])

        def wait_ring(slot):
            pltpu.make_async_copy(tbl_hbm.at[pl.ds(0, gr)],
                                  ring[slot], ring_sems[slot]).wait()

        def wait_flush(ab):
            pltpu.make_async_copy(accs[ab], out_hbm.at[pl.ds(0, FLUSH)],
                                  out_sems[ab]).wait()

        for g in range(NBUF):
            fire(g, g)

        def q_body(qo, _):
            g_base = qo * q_g
            for t in range(q_g):
                slot = t % NBUF
                ab = t // g_per_flush
                if t % g_per_flush == 0:
                    @pl.when(qo >= 1)
                    def _():
                        wait_flush(ab)
                wait_ring(slot)
                buf = ring[slot]
                acc = accs[ab]
                row0 = (t % g_per_flush) * SPG

                def samp_body(si, _):
                    r = si * seq
                    vals = [buf[r, pl.ds(k * LANES, LANES)]
                            for k in range(n_col)]
                    for l in range(1, seq):
                        for k in range(n_col):
                            vals[k] = vals[k] + buf[r + l,
                                                    pl.ds(k * LANES, LANES)]
                    for k in range(n_col):
                        acc[row0 + si, pl.ds(k * LANES, LANES)] = vals[k]
                    return 0

                lax.fori_loop(0, SPG, samp_body, 0)
                if t < q_g - NBUF:
                    fire(g_base + t + NBUF, slot)
                else:
                    @pl.when(qo < n_q - 1)
                    def _():
                        fire(g_base + t + NBUF, slot)
                if t % g_per_flush == g_per_flush - 1:
                    blk = 2 * qo + ab
                    pltpu.async_copy(
                        acc,
                        out_hbm.at[pl.ds(out_base + blk * FLUSH, FLUSH)],
                        out_sems[ab])
            return 0

        lax.fori_loop(0, n_q, q_body, 0)
        wait_flush(0)
        wait_flush(1)

    return pool


def _classify_body(pooled_ref, w_ref, b_ref, out_ref):
    out_ref[...] = lax.dot_general(
        w_ref[...], pooled_ref[...], (((1,), (1,)), ((), ())),
        preferred_element_type=jnp.float32) + b_ref[...]


def _classify(pooled, w2, bc):
    batch, d = pooled.shape
    nc = w2.shape[0]
    blk = 8192
    return pl.pallas_call(
        _classify_body,
        grid=(batch // blk,),
        in_specs=[
            pl.BlockSpec((blk, d), lambda i: (i, 0)),
            pl.BlockSpec((nc, d), lambda i: (0, 0)),
            pl.BlockSpec((nc, 1), lambda i: (0, 0)),
        ],
        out_specs=pl.BlockSpec((nc, blk), lambda i: (0, i)),
        out_shape=jax.ShapeDtypeStruct((nc, batch), jnp.float32),
    )(pooled, w2, bc)


def kernel(x, table, W, b):
    batch, seq = x.shape
    num_classes, d = W.shape
    pool = _make_sc_pool(batch, seq, d)
    pooled = pool(table, x)
    w2 = W * (1.0 / seq)
    yt = _classify(pooled, w2, b.reshape(num_classes, 1))
    return yt.T


# speedup vs baseline: 1.0045x; 1.0045x over previous
import functools

import jax
import jax.numpy as jnp
from jax import lax
from jax.experimental import pallas as pl
from jax.experimental.pallas import tpu as pltpu
from jax.experimental.pallas import tpu_sc as plsc

NC = 2
NS = 16
NW = NC * NS

LANES = 16
NBUF = 2
SPG = 16
FLUSH = 32
STG = 128


def _make_sc_pool(batch, seq, d):
    spw = batch // NW
    gr = SPG * seq
    n_g = spw // SPG
    g_per_flush = FLUSH // SPG
    n_col = d // LANES
    assert n_g % (2 * g_per_flush) == 0
    q_g = 2 * g_per_flush
    n_q = n_g // q_g
    mesh = plsc.VectorSubcoreMesh(core_axis_name="c", subcore_axis_name="s")

    @functools.partial(
        pl.kernel,
        mesh=mesh,
        out_type=jax.ShapeDtypeStruct((batch, d), jnp.float32),
        scratch_types=[
            pltpu.VMEM((spw * seq,), jnp.int32),
            pltpu.VMEM((STG, seq), jnp.int32),
            [pltpu.VMEM((gr, d), jnp.float32) for _ in range(NBUF)],
            [pltpu.VMEM((FLUSH, d), jnp.float32) for _ in range(2)],
            [pltpu.SemaphoreType.DMA for _ in range(NBUF)],
            [pltpu.SemaphoreType.DMA for _ in range(2)],
        ],
    )
    def pool(tbl_hbm, x_hbm, out_hbm, idx_v, stg_v, ring, accs, ring_sems,
             out_sems):
        wid = lax.axis_index("s") * NC + lax.axis_index("c")
        out_base = wid * spw

        def stage_body(part, _):
            pltpu.sync_copy(
                x_hbm.at[pl.ds(out_base + part * STG, STG), :], stg_v)

            def repack_body(r, _):
                p = (part * STG + r) * seq
                v0 = stg_v[r, pl.ds(0, LANES)]
                v1 = stg_v[r, pl.ds(seq - LANES, LANES)]
                idx_v[pl.ds(p, LANES)] = v0
                idx_v[pl.ds(p + seq - LANES, LANES)] = v1
                return 0

            lax.fori_loop(0, STG, repack_body, 0)
            return 0

        lax.fori_loop(0, spw // STG, stage_body, 0)

        def fire(g, slot):
            pltpu.async_copy(
                tbl_hbm.at[idx_v.at[pl.ds(g * gr, gr)]],
                ring[slot], ring_sems[slot])

        def wait_ring(slot):
            pltpu.make_async_copy(tbl_hbm.at[pl.ds(0, gr)],
                                  ring[slot], ring_sems[slot]).wait()

        def wait_flush(ab):
            pltpu.make_async_copy(accs[ab], out_hbm.at[pl.ds(0, FLUSH)],
                                  out_sems[ab]).wait()

        for g in range(NBUF):
            fire(g, g)

        def q_body(qo, _):
            g_base = qo * q_g
            for t in range(q_g):
                slot = t % NBUF
                ab = t // g_per_flush
                if t % g_per_flush == 0:
                    @pl.when(qo >= 1)
                    def _():
                        wait_flush(ab)
                wait_ring(slot)
                buf = ring[slot]
                acc = accs[ab]
                row0 = (t % g_per_flush) * SPG

                def samp_body(si, _):
                    r = si * seq
                    vals = [buf[r, pl.ds(k * LANES, LANES)]
                            for k in range(n_col)]
                    for l in range(1, seq):
                        for k in range(n_col):
                            vals[k] = vals[k] + buf[r + l,
                                                    pl.ds(k * LANES, LANES)]
                    for k in range(n_col):
                        acc[row0 + si, pl.ds(k * LANES, LANES)] = vals[k]
                    return 0

                lax.fori_loop(0, SPG, samp_body, 0)
                if t < q_g - NBUF:
                    fire(g_base + t + NBUF, slot)
                else:
                    @pl.when(qo < n_q - 1)
                    def _():
                        fire(g_base + t + NBUF, slot)
                if t % g_per_flush == g_per_flush - 1:
                    blk = 2 * qo + ab
                    pltpu.async_copy(
                        acc,
                        out_hbm.at[pl.ds(out_base + blk * FLUSH, FLUSH)],
                        out_sems[ab])
            return 0

        lax.fori_loop(0, n_q, q_body, 0)
        wait_flush(0)
        wait_flush(1)

    return pool


def _classify_body(pooled_ref, w_ref, b_ref, out_ref):
    out_ref[...] = lax.dot_general(
        w_ref[...], pooled_ref[...], (((1,), (1,)), ((), ())),
        preferred_element_type=jnp.float32) + b_ref[...]


def _classify(pooled, w2, bc):
    batch, d = pooled.shape
    nc = w2.shape[0]
    blk = 8192
    return pl.pallas_call(
        _classify_body,
        grid=(batch // blk,),
        in_specs=[
            pl.BlockSpec((blk, d), lambda i: (i, 0)),
            pl.BlockSpec((nc, d), lambda i: (0, 0)),
            pl.BlockSpec((nc, 1), lambda i: (0, 0)),
        ],
        out_specs=pl.BlockSpec((nc, blk), lambda i: (0, i)),
        out_shape=jax.ShapeDtypeStruct((nc, batch), jnp.float32),
    )(pooled, w2, bc)


def kernel(x, table, W, b):
    batch, seq = x.shape
    num_classes, d = W.shape
    pool = _make_sc_pool(batch, seq, d)
    pooled = pool(table, x)
    w2 = W * (1.0 / seq)
    yt = _classify(pooled, w2, b.reshape(num_classes, 1))
    return yt.T


# speedup vs baseline: 1.0954x; 1.0905x over previous
import functools

import jax
import jax.numpy as jnp
from jax import lax
from jax.experimental import pallas as pl
from jax.experimental.pallas import tpu as pltpu
from jax.experimental.pallas import tpu_sc as plsc

NC = 2
NS = 16
NW = NC * NS

LANES = 16
NBUF = 4
SPG = 8
FLUSH = 32
STG = 128


def _make_sc_pool(batch, seq, d):
    spw = batch // NW
    gr = SPG * seq
    n_g = spw // SPG
    g_per_flush = FLUSH // SPG
    n_col = d // LANES
    assert n_g % (2 * g_per_flush) == 0
    q_g = 2 * g_per_flush
    n_q = n_g // q_g
    mesh = plsc.VectorSubcoreMesh(core_axis_name="c", subcore_axis_name="s")

    @functools.partial(
        pl.kernel,
        mesh=mesh,
        out_type=jax.ShapeDtypeStruct((batch, d), jnp.float32),
        scratch_types=[
            pltpu.VMEM((spw * seq,), jnp.int32),
            pltpu.VMEM((STG, seq), jnp.int32),
            [pltpu.VMEM((gr, d), jnp.float32) for _ in range(NBUF)],
            [pltpu.VMEM((FLUSH, d), jnp.float32) for _ in range(2)],
            [pltpu.SemaphoreType.DMA for _ in range(NBUF)],
            [pltpu.SemaphoreType.DMA for _ in range(2)],
        ],
    )
    def pool(tbl_hbm, x_hbm, out_hbm, idx_v, stg_v, ring, accs, ring_sems,
             out_sems):
        wid = lax.axis_index("s") * NC + lax.axis_index("c")
        out_base = wid * spw

        def stage_body(part, _):
            pltpu.sync_copy(
                x_hbm.at[pl.ds(out_base + part * STG, STG), :], stg_v)

            def repack_body(r, _):
                p = (part * STG + r) * seq
                v0 = stg_v[r, pl.ds(0, LANES)]
                v1 = stg_v[r, pl.ds(seq - LANES, LANES)]
                idx_v[pl.ds(p, LANES)] = v0
                idx_v[pl.ds(p + seq - LANES, LANES)] = v1
                return 0

            lax.fori_loop(0, STG, repack_body, 0)
            return 0

        def fire(g, slot):
            pltpu.async_copy(
                tbl_hbm.at[idx_v.at[pl.ds(g * gr, gr)]],
                ring[slot], ring_sems[slot])

        def wait_ring(slot):
            pltpu.make_async_copy(tbl_hbm.at[pl.ds(0, gr)],
                                  ring[slot], ring_sems[slot]).wait()

        def wait_flush(ab):
            pltpu.make_async_copy(accs[ab], out_hbm.at[pl.ds(0, FLUSH)],
                                  out_sems[ab]).wait()

        stage_body(0, 0)
        for g in range(NBUF):
            fire(g, g)
        lax.fori_loop(1, spw // STG, stage_body, 0)

        def q_body(qo, _):
            g_base = qo * q_g
            for t in range(q_g):
                slot = t % NBUF
                ab = t // g_per_flush
                if t % g_per_flush == 0:
                    @pl.when(qo >= 1)
                    def _():
                        wait_flush(ab)
                wait_ring(slot)
                buf = ring[slot]
                acc = accs[ab]
                row0 = (t % g_per_flush) * SPG

                def samp_body(si, _):
                    r = si * seq
                    vals = [buf[r, pl.ds(k * LANES, LANES)]
                            for k in range(n_col)]
                    for l in range(1, seq):
                        for k in range(n_col):
                            vals[k] = vals[k] + buf[r + l,
                                                    pl.ds(k * LANES, LANES)]
                    for k in range(n_col):
                        acc[row0 + si, pl.ds(k * LANES, LANES)] = vals[k]
                    return 0

                lax.fori_loop(0, SPG, samp_body, 0)
                if t < q_g - NBUF:
                    fire(g_base + t + NBUF, slot)
                else:
                    @pl.when(qo < n_q - 1)
                    def _():
                        fire(g_base + t + NBUF, slot)
                if t % g_per_flush == g_per_flush - 1:
                    blk = 2 * qo + ab
                    pltpu.async_copy(
                        acc,
                        out_hbm.at[pl.ds(out_base + blk * FLUSH, FLUSH)],
                        out_sems[ab])
            return 0

        lax.fori_loop(0, n_q, q_body, 0)
        wait_flush(0)
        wait_flush(1)

    return pool


def _classify_body(pooled_ref, w_ref, b_ref, out_ref):
    out_ref[...] = lax.dot_general(
        w_ref[...], pooled_ref[...], (((1,), (1,)), ((), ())),
        preferred_element_type=jnp.float32) + b_ref[...]


def _classify(pooled, w2, bc):
    batch, d = pooled.shape
    nc = w2.shape[0]
    blk = 8192
    return pl.pallas_call(
        _classify_body,
        grid=(batch // blk,),
        in_specs=[
            pl.BlockSpec((blk, d), lambda i: (i, 0)),
            pl.BlockSpec((nc, d), lambda i: (0, 0)),
            pl.BlockSpec((nc, 1), lambda i: (0, 0)),
        ],
        out_specs=pl.BlockSpec((nc, blk), lambda i: (0, i)),
        out_shape=jax.ShapeDtypeStruct((nc, batch), jnp.float32),
    )(pooled, w2, bc)


def kernel(x, table, W, b):
    batch, seq = x.shape
    num_classes, d = W.shape
    pool = _make_sc_pool(batch, seq, d)
    pooled = pool(table, x)
    w2 = W * (1.0 / seq)
    yt = _classify(pooled, w2, b.reshape(num_classes, 1))
    return yt.T
